# baseline (speedup 1.0000x reference)
import functools

import jax
import jax.numpy as jnp
from jax import lax
from jax.experimental import pallas as pl
from jax.experimental.pallas import tpu as pltpu
from jax.experimental.pallas import tpu_sc as plsc

N = 50000
E = 800000
D = 64
HALF = N // 2
G = 80
EP = E
EPT = EP // 16
CE = 2000
CI = CE // G
NOUT = EPT // CE
TRASH = 25088
ACC_R = TRASH + G
ZROWS = 1568
ZLAST = HALF - 15 * ZROWS
NBUF = 4
KAH = 2

_mesh = plsc.VectorSubcoreMesh(core_axis_name="c", subcore_axis_name="s")


@functools.partial(
    pl.kernel,
    out_type=jax.ShapeDtypeStruct((N, 8), jnp.float32),
    mesh=_mesh,
    compiler_params=pltpu.CompilerParams(use_tc_tiling_on_sc=False),
    scratch_types=[
        pltpu.VMEM((2, CI, G), jnp.int32),
        pltpu.VMEM((G, 8), jnp.float32),
        pltpu.VMEM_SHARED((ACC_R, 8), jnp.float32),
        pltpu.SemaphoreType.DMA,
        pltpu.SemaphoreType.DMA,
    ],
)
def _degcount(dst2, zb8, ones8, deg8, dstb, onesb, acc8, dsem, stsem):
    c = lax.axis_index("c")
    s = lax.axis_index("s")
    lo = c * HALF
    pltpu.sync_copy(ones8, onesb)

    @pl.when(s < 15)
    def _zero_main():
        pltpu.sync_copy(zb8, acc8.at[pl.ds(s * ZROWS, ZROWS)])

    @pl.when(s == 15)
    def _zero_last():
        pltpu.sync_copy(zb8.at[pl.ds(0, ZLAST)],
                        acc8.at[pl.ds(15 * ZROWS, ZLAST)])

    plsc.subcore_barrier()

    base = s * EPT

    def _stage(i, ib):
        pltpu.async_copy(dst2.at[pl.ds((base + i * CE) // G, CI)],
                         dstb.at[ib], stsem)

    _stage(0, 0)

    def outer(i, carry):
        ib = lax.rem(i, 2)
        pltpu.make_async_copy(dst2.at[pl.ds(0, CI)], dstb.at[ib],
                              stsem).wait()

        @pl.when(i + 1 < NOUT)
        def _stage_next():
            _stage(i + 1, 1 - ib)

        def comp(j, carry2):
            for k in range(G // 16):
                d = dstb[ib, j, pl.ds(k * 16, 16)]
                keep = (d >= lo) & (d < lo + HALF)
                trash = TRASH + k * 16 + lax.iota(jnp.int32, 16)
                dstb[ib, j, pl.ds(k * 16, 16)] = jnp.where(keep, d - lo,
                                                           trash)
            return carry2

        lax.fori_loop(0, CI, comp, 0)

        sd = [None] * CI
        for j in range(CI):
            sd[j] = pltpu.async_copy(onesb, acc8.at[dstb.at[ib, j]], dsem,
                                     add=True)
            if j >= 6:
                sd[j - 6].wait()
        for j in range(max(CI - 6, 0), CI):
            sd[j].wait()
        return carry

    lax.fori_loop(0, NOUT, outer, 0)
    plsc.subcore_barrier()

    @pl.when(s < 15)
    def _out_main():
        pltpu.sync_copy(acc8.at[pl.ds(s * ZROWS, ZROWS)],
                        deg8.at[pl.ds(c * HALF + s * ZROWS, ZROWS)])

    @pl.when(s == 15)
    def _out_last():
        pltpu.sync_copy(acc8.at[pl.ds(15 * ZROWS, ZLAST)],
                        deg8.at[pl.ds(c * HALF + 15 * ZROWS, ZLAST)])


SCE = 2000
SCI = SCE // G
NGRP = EPT // G
NCH = EPT // SCE


@functools.partial(
    pl.kernel,
    out_type=jax.ShapeDtypeStruct((N, D), jnp.float32),
    mesh=_mesh,
    compiler_params=pltpu.CompilerParams(use_tc_tiling_on_sc=False),
    scratch_types=[
        pltpu.VMEM((2, SCE), jnp.int32),
        pltpu.VMEM((2, SCI, G), jnp.int32),
        pltpu.VMEM((NBUF, G, D), jnp.float32),
        pltpu.VMEM_SHARED((ACC_R, D), jnp.float32),
        pltpu.SemaphoreType.DMA((NBUF,)),
        pltpu.SemaphoreType.DMA((NBUF,)),
        pltpu.SemaphoreType.DMA,
    ],
)
def _segsum(h, src, dst2, zb, seg, srcb, dstb, rows, acc, gsem, ssem, stsem):
    c = lax.axis_index("c")
    s = lax.axis_index("s")
    lo = c * HALF

    @pl.when(s < 15)
    def _zero_main():
        pltpu.sync_copy(zb, acc.at[pl.ds(s * ZROWS, ZROWS)])

    @pl.when(s == 15)
    def _zero_last():
        pltpu.sync_copy(zb.at[pl.ds(0, ZLAST)],
                        acc.at[pl.ds(15 * ZROWS, ZLAST)])

    plsc.subcore_barrier()

    base = s * EPT

    def _stage(i, ib):
        e0 = base + i * SCE
        pltpu.async_copy(src.at[pl.ds(e0, SCE)], srcb.at[ib], stsem)
        pltpu.async_copy(dst2.at[pl.ds(e0 // G, SCI)], dstb.at[ib], stsem)

    _stage(0, 0)

    def outer(i, carry):
        ib = lax.rem(i, 2)
        pltpu.make_async_copy(src.at[pl.ds(0, SCE)], srcb.at[ib],
                              stsem).wait()
        pltpu.make_async_copy(dst2.at[pl.ds(0, SCI)], dstb.at[ib],
                              stsem).wait()

        @pl.when(i + 1 < NCH)
        def _stage_next():
            _stage(i + 1, 1 - ib)

        def comp(j, carry2):
            for k in range(G // 16):
                d = dstb[ib, j, pl.ds(k * 16, 16)]
                keep = (d >= lo) & (d < lo + HALF)
                trash = TRASH + k * 16 + lax.iota(jnp.int32, 16)
                dstb[ib, j, pl.ds(k * 16, 16)] = jnp.where(keep, d - lo,
                                                           trash)
            return carry2

        lax.fori_loop(0, SCI, comp, 0)

        gd = [None] * SCI
        sd = [None] * SCI
        for j in range(KAH):
            gd[j] = pltpu.async_copy(
                h.at[srcb.at[ib, pl.ds(j * G, G)]], rows.at[j % NBUF],
                gsem.at[j % NBUF])
        for j in range(SCI):
            b = j % NBUF
            gd[j].wait()
            if j >= NBUF - KAH:
                sd[j - (NBUF - KAH)].wait()
            jn = j + KAH
            if jn < SCI:
                gd[jn] = pltpu.async_copy(
                    h.at[srcb.at[ib, pl.ds(jn * G, G)]], rows.at[jn % NBUF],
                    gsem.at[jn % NBUF])
            sd[j] = pltpu.async_copy(rows.at[b], acc.at[dstb.at[ib, j]],
                                     ssem.at[b], add=True)
        for j in range(max(SCI - (NBUF - KAH), 0), SCI):
            sd[j].wait()
        return carry

    lax.fori_loop(0, NCH, outer, 0)
    plsc.subcore_barrier()

    @pl.when(s < 15)
    def _out_main():
        pltpu.sync_copy(acc.at[pl.ds(s * ZROWS, ZROWS)],
                        seg.at[pl.ds(c * HALF + s * ZROWS, ZROWS)])

    @pl.when(s == 15)
    def _out_last():
        pltpu.sync_copy(acc.at[pl.ds(15 * ZROWS, ZLAST)],
                        seg.at[pl.ds(c * HALF + 15 * ZROWS, ZLAST)])


BN = 2000


def _embed_body(x_ref, w_ref, b_ref, o_ref):
    o_ref[...] = (
        jnp.dot(x_ref[...], w_ref[...], preferred_element_type=jnp.float32)
        + b_ref[...]
    )


def _embed(x, W_in, b_in):
    return pl.pallas_call(
        _embed_body,
        grid=(N // BN,),
        in_specs=[
            pl.BlockSpec((BN, 2), lambda i: (i, 0)),
            pl.BlockSpec((2, D), lambda i: (0, 0)),
            pl.BlockSpec((1, D), lambda i: (0, 0)),
        ],
        out_specs=pl.BlockSpec((BN, D), lambda i: (i, 0)),
        out_shape=jax.ShapeDtypeStruct((N, D), jnp.float32),
    )(x, W_in, b_in)


def _dense_body(h_ref, seg_ref, deg_ref, ws_ref, wn_ref, b_ref, o_ref):
    h = h_ref[...]
    deg = jnp.maximum(deg_ref[:, 0:1], 1.0)
    msg = seg_ref[...] / deg
    z = (
        jnp.dot(h, ws_ref[...], preferred_element_type=jnp.float32)
        + jnp.dot(msg, wn_ref[...], preferred_element_type=jnp.float32)
        + b_ref[...]
    )
    o_ref[...] = h + jnp.where(z >= 0, z, 0.01 * z)


def _dense(h, seg, deg8, Ws, Wn, bias):
    return pl.pallas_call(
        _dense_body,
        grid=(N // BN,),
        in_specs=[
            pl.BlockSpec((BN, D), lambda i: (i, 0)),
            pl.BlockSpec((BN, D), lambda i: (i, 0)),
            pl.BlockSpec((BN, 8), lambda i: (i, 0)),
            pl.BlockSpec((D, D), lambda i: (0, 0)),
            pl.BlockSpec((D, D), lambda i: (0, 0)),
            pl.BlockSpec((1, D), lambda i: (0, 0)),
        ],
        out_specs=pl.BlockSpec((BN, D), lambda i: (i, 0)),
        out_shape=jax.ShapeDtypeStruct((N, D), jnp.float32),
    )(h, seg, deg8, Ws, Wn, bias)


def kernel(x, edge_index, W_in, b_in, Wself, Wnei, b):
    src = edge_index[0]
    dst2 = edge_index[1].reshape(EP // G, G)
    zb = jnp.zeros((ZROWS, D), jnp.float32)
    zb8 = jnp.zeros((ZROWS, 8), jnp.float32)
    ones8 = jnp.ones((G, 8), jnp.float32)

    h = _embed(x, W_in, b_in.reshape(1, D))
    deg8 = _degcount(dst2, zb8, ones8)
    for l in range(3):
        seg = _segsum(h, src, dst2, zb)
        h = _dense(h, seg, deg8, Wself[l], Wnei[l], b[l].reshape(1, D))
    return h


# speedup vs baseline: 1.2300x; 1.2300x over previous
import functools

import jax
import jax.numpy as jnp
from jax import lax
from jax.experimental import pallas as pl
from jax.experimental.pallas import tpu as pltpu
from jax.experimental.pallas import tpu_sc as plsc

N = 50000
E = 800000
D = 64
HALF = N // 2
G = 80
EP = E
EPT = EP // 16
CE = 2000
CI = CE // G
NOUT = EPT // CE
TRASH = 25088
ACC_R = TRASH + G
ZROWS = 1568
ZLAST = HALF - 15 * ZROWS
NBUF = 4
KAH = 3

_mesh = plsc.VectorSubcoreMesh(core_axis_name="c", subcore_axis_name="s")


@functools.partial(
    pl.kernel,
    out_type=jax.ShapeDtypeStruct((N, 8), jnp.float32),
    mesh=_mesh,
    compiler_params=pltpu.CompilerParams(use_tc_tiling_on_sc=False),
    scratch_types=[
        pltpu.VMEM((2, CI, G), jnp.int32),
        pltpu.VMEM((G, 8), jnp.float32),
        pltpu.VMEM_SHARED((ACC_R, 8), jnp.float32),
        pltpu.SemaphoreType.DMA,
        pltpu.SemaphoreType.DMA,
    ],
)
def _degcount(dst2, zb8, ones8, deg8, dstb, onesb, acc8, dsem, stsem):
    c = lax.axis_index("c")
    s = lax.axis_index("s")
    lo = c * HALF
    pltpu.sync_copy(ones8, onesb)

    @pl.when(s < 15)
    def _zero_main():
        pltpu.sync_copy(zb8, acc8.at[pl.ds(s * ZROWS, ZROWS)])

    @pl.when(s == 15)
    def _zero_last():
        pltpu.sync_copy(zb8.at[pl.ds(0, ZLAST)],
                        acc8.at[pl.ds(15 * ZROWS, ZLAST)])

    plsc.subcore_barrier()

    base = s * EPT

    def _stage(i, ib):
        pltpu.async_copy(dst2.at[pl.ds((base + i * CE) // G, CI)],
                         dstb.at[ib], stsem)

    _stage(0, 0)

    def outer(i, carry):
        ib = lax.rem(i, 2)
        pltpu.make_async_copy(dst2.at[pl.ds(0, CI)], dstb.at[ib],
                              stsem).wait()

        @pl.when(i + 1 < NOUT)
        def _stage_next():
            _stage(i + 1, 1 - ib)

        def comp(j, carry2):
            for k in range(G // 16):
                d = dstb[ib, j, pl.ds(k * 16, 16)]
                keep = (d >= lo) & (d < lo + HALF)
                trash = TRASH + k * 16 + lax.iota(jnp.int32, 16)
                dstb[ib, j, pl.ds(k * 16, 16)] = jnp.where(keep, d - lo,
                                                           trash)
            return carry2

        lax.fori_loop(0, CI, comp, 0)

        sd = [None] * CI
        for j in range(CI):
            sd[j] = pltpu.async_copy(onesb, acc8.at[dstb.at[ib, j]], dsem,
                                     add=True)
            if j >= 6:
                sd[j - 6].wait()
        for j in range(max(CI - 6, 0), CI):
            sd[j].wait()
        return carry

    lax.fori_loop(0, NOUT, outer, 0)
    plsc.subcore_barrier()

    @pl.when(s < 15)
    def _out_main():
        pltpu.sync_copy(acc8.at[pl.ds(s * ZROWS, ZROWS)],
                        deg8.at[pl.ds(c * HALF + s * ZROWS, ZROWS)])

    @pl.when(s == 15)
    def _out_last():
        pltpu.sync_copy(acc8.at[pl.ds(15 * ZROWS, ZLAST)],
                        deg8.at[pl.ds(c * HALF + 15 * ZROWS, ZLAST)])


SCE = 2000
SCI = SCE // G
NGRP = EPT // G
NCH = EPT // SCE


@functools.partial(
    pl.kernel,
    out_type=jax.ShapeDtypeStruct((N, D), jnp.float32),
    mesh=_mesh,
    compiler_params=pltpu.CompilerParams(use_tc_tiling_on_sc=False),
    scratch_types=[
        pltpu.VMEM((2, SCE), jnp.int32),
        pltpu.VMEM((2, SCI, G), jnp.int32),
        pltpu.VMEM((NBUF, G, D), jnp.float32),
        pltpu.VMEM_SHARED((ACC_R, D), jnp.float32),
        pltpu.SemaphoreType.DMA((NBUF,)),
        pltpu.SemaphoreType.DMA((NBUF,)),
        pltpu.SemaphoreType.DMA,
    ],
)
def _segsum(h, src, dst2, zb, seg, srcb, dstb, rows, acc, gsem, ssem, stsem):
    c = lax.axis_index("c")
    s = lax.axis_index("s")
    lo = c * HALF

    @pl.when(s < 15)
    def _zero_main():
        pltpu.sync_copy(zb, acc.at[pl.ds(s * ZROWS, ZROWS)])

    @pl.when(s == 15)
    def _zero_last():
        pltpu.sync_copy(zb.at[pl.ds(0, ZLAST)],
                        acc.at[pl.ds(15 * ZROWS, ZLAST)])

    plsc.subcore_barrier()

    base = s * EPT

    def _stage(i, ib):
        e0 = base + i * SCE
        pltpu.async_copy(src.at[pl.ds(e0, SCE)], srcb.at[ib], stsem)
        pltpu.async_copy(dst2.at[pl.ds(e0 // G, SCI)], dstb.at[ib], stsem)

    _stage(0, 0)

    def outer(i, carry):
        ib = lax.rem(i, 2)
        pltpu.make_async_copy(src.at[pl.ds(0, SCE)], srcb.at[ib],
                              stsem).wait()
        pltpu.make_async_copy(dst2.at[pl.ds(0, SCI)], dstb.at[ib],
                              stsem).wait()

        @pl.when(i + 1 < NCH)
        def _stage_next():
            _stage(i + 1, 1 - ib)

        def comp(j, carry2):
            for k in range(G // 16):
                d = dstb[ib, j, pl.ds(k * 16, 16)]
                keep = (d >= lo) & (d < lo + HALF)
                trash = TRASH + k * 16 + lax.iota(jnp.int32, 16)
                dstb[ib, j, pl.ds(k * 16, 16)] = jnp.where(keep, d - lo,
                                                           trash)
            return carry2

        lax.fori_loop(0, SCI, comp, 0)

        gd = [None] * SCI
        sd = [None] * SCI
        for j in range(KAH):
            gd[j] = pltpu.async_copy(
                h.at[srcb.at[ib, pl.ds(j * G, G)]], rows.at[j % NBUF],
                gsem.at[j % NBUF])
        for j in range(SCI):
            b = j % NBUF
            gd[j].wait()
            if j >= NBUF - KAH:
                sd[j - (NBUF - KAH)].wait()
            jn = j + KAH
            if jn < SCI:
                gd[jn] = pltpu.async_copy(
                    h.at[srcb.at[ib, pl.ds(jn * G, G)]], rows.at[jn % NBUF],
                    gsem.at[jn % NBUF])
            sd[j] = pltpu.async_copy(rows.at[b], acc.at[dstb.at[ib, j]],
                                     ssem.at[b], add=True)
        for j in range(max(SCI - (NBUF - KAH), 0), SCI):
            sd[j].wait()
        return carry

    lax.fori_loop(0, NCH, outer, 0)
    plsc.subcore_barrier()

    @pl.when(s < 15)
    def _out_main():
        pltpu.sync_copy(acc.at[pl.ds(s * ZROWS, ZROWS)],
                        seg.at[pl.ds(c * HALF + s * ZROWS, ZROWS)])

    @pl.when(s == 15)
    def _out_last():
        pltpu.sync_copy(acc.at[pl.ds(15 * ZROWS, ZLAST)],
                        seg.at[pl.ds(c * HALF + 15 * ZROWS, ZLAST)])


BN = 2000


def _embed_body(x_ref, w_ref, b_ref, o_ref):
    o_ref[...] = (
        jnp.dot(x_ref[...], w_ref[...], preferred_element_type=jnp.float32)
        + b_ref[...]
    )


def _embed(x, W_in, b_in):
    return pl.pallas_call(
        _embed_body,
        grid=(N // BN,),
        in_specs=[
            pl.BlockSpec((BN, 2), lambda i: (i, 0)),
            pl.BlockSpec((2, D), lambda i: (0, 0)),
            pl.BlockSpec((1, D), lambda i: (0, 0)),
        ],
        out_specs=pl.BlockSpec((BN, D), lambda i: (i, 0)),
        out_shape=jax.ShapeDtypeStruct((N, D), jnp.float32),
    )(x, W_in, b_in)


def _dense_body(h_ref, seg_ref, deg_ref, ws_ref, wn_ref, b_ref, o_ref):
    h = h_ref[...]
    deg = jnp.maximum(deg_ref[:, 0:1], 1.0)
    msg = seg_ref[...] / deg
    z = (
        jnp.dot(h, ws_ref[...], preferred_element_type=jnp.float32)
        + jnp.dot(msg, wn_ref[...], preferred_element_type=jnp.float32)
        + b_ref[...]
    )
    o_ref[...] = h + jnp.where(z >= 0, z, 0.01 * z)


def _dense(h, seg, deg8, Ws, Wn, bias):
    return pl.pallas_call(
        _dense_body,
        grid=(N // BN,),
        in_specs=[
            pl.BlockSpec((BN, D), lambda i: (i, 0)),
            pl.BlockSpec((BN, D), lambda i: (i, 0)),
            pl.BlockSpec((BN, 8), lambda i: (i, 0)),
            pl.BlockSpec((D, D), lambda i: (0, 0)),
            pl.BlockSpec((D, D), lambda i: (0, 0)),
            pl.BlockSpec((1, D), lambda i: (0, 0)),
        ],
        out_specs=pl.BlockSpec((BN, D), lambda i: (i, 0)),
        out_shape=jax.ShapeDtypeStruct((N, D), jnp.float32),
    )(h, seg, deg8, Ws, Wn, bias)


def kernel(x, edge_index, W_in, b_in, Wself, Wnei, b):
    src = edge_index[0]
    dst2 = edge_index[1].reshape(EP // G, G)
    zb = jnp.zeros((ZROWS, D), jnp.float32)
    zb8 = jnp.zeros((ZROWS, 8), jnp.float32)
    ones8 = jnp.ones((G, 8), jnp.float32)

    h = _embed(x, W_in, b_in.reshape(1, D))
    deg8 = _degcount(dst2, zb8, ones8)
    for l in range(3):
        seg = _segsum(h, src, dst2, zb)
        h = _dense(h, seg, deg8, Wself[l], Wnei[l], b[l].reshape(1, D))
    return h


# speedup vs baseline: 1.2625x; 1.0264x over previous
import functools

import jax
import jax.numpy as jnp
from jax import lax
from jax.experimental import pallas as pl
from jax.experimental.pallas import tpu as pltpu
from jax.experimental.pallas import tpu_sc as plsc

N = 50000
E = 800000
D = 64
HALF = N // 2
G = 80
EP = E
EPT = EP // 16
CE = 2000
CI = CE // G
NOUT = EPT // CE
TRASH = 25088
ACC_R = TRASH + G
ZROWS = 1568
ZLAST = HALF - 15 * ZROWS
NBUF = 4
KAH = 3

_mesh = plsc.VectorSubcoreMesh(core_axis_name="c", subcore_axis_name="s")


@functools.partial(
    pl.kernel,
    out_type=jax.ShapeDtypeStruct((N, 8), jnp.float32),
    mesh=_mesh,
    compiler_params=pltpu.CompilerParams(use_tc_tiling_on_sc=False),
    scratch_types=[
        pltpu.VMEM((2, CI, G), jnp.int32),
        pltpu.VMEM((G, 8), jnp.float32),
        pltpu.VMEM_SHARED((ACC_R, 8), jnp.float32),
        pltpu.SemaphoreType.DMA,
        pltpu.SemaphoreType.DMA,
    ],
)
def _degcount(dst2, zb8, ones8, deg8, dstb, onesb, acc8, dsem, stsem):
    c = lax.axis_index("c")
    s = lax.axis_index("s")
    lo = c * HALF
    pltpu.sync_copy(ones8, onesb)

    @pl.when(s < 15)
    def _zero_main():
        pltpu.sync_copy(zb8, acc8.at[pl.ds(s * ZROWS, ZROWS)])

    @pl.when(s == 15)
    def _zero_last():
        pltpu.sync_copy(zb8.at[pl.ds(0, ZLAST)],
                        acc8.at[pl.ds(15 * ZROWS, ZLAST)])

    plsc.subcore_barrier()

    base = s * EPT

    def _stage(i, ib):
        pltpu.async_copy(dst2.at[pl.ds((base + i * CE) // G, CI)],
                         dstb.at[ib], stsem)

    _stage(0, 0)

    def outer(i, carry):
        ib = lax.rem(i, 2)
        pltpu.make_async_copy(dst2.at[pl.ds(0, CI)], dstb.at[ib],
                              stsem).wait()

        @pl.when(i + 1 < NOUT)
        def _stage_next():
            _stage(i + 1, 1 - ib)

        def comp(j, carry2):
            for k in range(G // 16):
                d = dstb[ib, j, pl.ds(k * 16, 16)]
                keep = (d >= lo) & (d < lo + HALF)
                trash = TRASH + k * 16 + lax.iota(jnp.int32, 16)
                dstb[ib, j, pl.ds(k * 16, 16)] = jnp.where(keep, d - lo,
                                                           trash)
            return carry2

        lax.fori_loop(0, CI, comp, 0)

        sd = [None] * CI
        for j in range(CI):
            sd[j] = pltpu.async_copy(onesb, acc8.at[dstb.at[ib, j]], dsem,
                                     add=True)
            if j >= 6:
                sd[j - 6].wait()
        for j in range(max(CI - 6, 0), CI):
            sd[j].wait()
        return carry

    lax.fori_loop(0, NOUT, outer, 0)
    plsc.subcore_barrier()

    @pl.when(s < 15)
    def _out_main():
        pltpu.sync_copy(acc8.at[pl.ds(s * ZROWS, ZROWS)],
                        deg8.at[pl.ds(c * HALF + s * ZROWS, ZROWS)])

    @pl.when(s == 15)
    def _out_last():
        pltpu.sync_copy(acc8.at[pl.ds(15 * ZROWS, ZLAST)],
                        deg8.at[pl.ds(c * HALF + 15 * ZROWS, ZLAST)])


SCE = 2000
SCI = SCE // G
NGRP = EPT // G
NCH = EPT // SCE


@functools.partial(
    pl.kernel,
    out_type=jax.ShapeDtypeStruct((N, D), jnp.float32),
    mesh=_mesh,
    compiler_params=pltpu.CompilerParams(use_tc_tiling_on_sc=False),
    scratch_types=[
        pltpu.VMEM((2, SCE), jnp.int32),
        pltpu.VMEM((2, SCI, G), jnp.int32),
        pltpu.VMEM((NBUF, G, D), jnp.float32),
        pltpu.VMEM_SHARED((ACC_R, D), jnp.float32),
        pltpu.SemaphoreType.DMA((NBUF,)),
        pltpu.SemaphoreType.DMA((NBUF,)),
        pltpu.SemaphoreType.DMA,
    ],
)
def _segsum(h, src, dst2, zb, seg, srcb, dstb, rows, acc, gsem, ssem, stsem):
    c = lax.axis_index("c")
    s = lax.axis_index("s")
    lo = c * HALF

    @pl.when(s < 15)
    def _zero_main():
        pltpu.sync_copy(zb, acc.at[pl.ds(s * ZROWS, ZROWS)])

    @pl.when(s == 15)
    def _zero_last():
        pltpu.sync_copy(zb.at[pl.ds(0, ZLAST)],
                        acc.at[pl.ds(15 * ZROWS, ZLAST)])

    plsc.subcore_barrier()

    base = s * EPT

    def _stage(i, ib):
        e0 = base + i * SCE
        pltpu.async_copy(src.at[pl.ds(e0, SCE)], srcb.at[ib], stsem)
        pltpu.async_copy(dst2.at[pl.ds(e0 // G, SCI)], dstb.at[ib], stsem)

    _stage(0, 0)

    def outer(i, carry):
        ib = lax.rem(i, 2)
        pltpu.make_async_copy(src.at[pl.ds(0, SCE)], srcb.at[ib],
                              stsem).wait()
        pltpu.make_async_copy(dst2.at[pl.ds(0, SCI)], dstb.at[ib],
                              stsem).wait()

        @pl.when(i + 1 < NCH)
        def _stage_next():
            _stage(i + 1, 1 - ib)

        gd = [None] * SCI
        sd = [None] * SCI
        for j in range(KAH):
            gd[j] = pltpu.async_copy(
                h.at[srcb.at[ib, pl.ds(j * G, G)]], rows.at[j % NBUF],
                gsem.at[j % NBUF])
        for j in range(SCI):
            b = j % NBUF
            for k in range(G // 16):
                d = dstb[ib, j, pl.ds(k * 16, 16)]
                keep = (d >= lo) & (d < lo + HALF)
                trash = TRASH + k * 16 + lax.iota(jnp.int32, 16)
                dstb[ib, j, pl.ds(k * 16, 16)] = jnp.where(keep, d - lo,
                                                           trash)
            gd[j].wait()
            if j >= NBUF - KAH:
                sd[j - (NBUF - KAH)].wait()
            jn = j + KAH
            if jn < SCI:
                gd[jn] = pltpu.async_copy(
                    h.at[srcb.at[ib, pl.ds(jn * G, G)]], rows.at[jn % NBUF],
                    gsem.at[jn % NBUF])
            sd[j] = pltpu.async_copy(rows.at[b], acc.at[dstb.at[ib, j]],
                                     ssem.at[b], add=True)
        for j in range(max(SCI - (NBUF - KAH), 0), SCI):
            sd[j].wait()
        return carry

    lax.fori_loop(0, NCH, outer, 0)
    plsc.subcore_barrier()

    @pl.when(s < 15)
    def _out_main():
        pltpu.sync_copy(acc.at[pl.ds(s * ZROWS, ZROWS)],
                        seg.at[pl.ds(c * HALF + s * ZROWS, ZROWS)])

    @pl.when(s == 15)
    def _out_last():
        pltpu.sync_copy(acc.at[pl.ds(15 * ZROWS, ZLAST)],
                        seg.at[pl.ds(c * HALF + 15 * ZROWS, ZLAST)])


BN = 2000


def _embed_body(x_ref, w_ref, b_ref, o_ref):
    o_ref[...] = (
        jnp.dot(x_ref[...], w_ref[...], preferred_element_type=jnp.float32)
        + b_ref[...]
    )


def _embed(x, W_in, b_in):
    return pl.pallas_call(
        _embed_body,
        grid=(N // BN,),
        in_specs=[
            pl.BlockSpec((BN, 2), lambda i: (i, 0)),
            pl.BlockSpec((2, D), lambda i: (0, 0)),
            pl.BlockSpec((1, D), lambda i: (0, 0)),
        ],
        out_specs=pl.BlockSpec((BN, D), lambda i: (i, 0)),
        out_shape=jax.ShapeDtypeStruct((N, D), jnp.float32),
    )(x, W_in, b_in)


def _dense_body(h_ref, seg_ref, deg_ref, ws_ref, wn_ref, b_ref, o_ref):
    h = h_ref[...]
    deg = jnp.maximum(deg_ref[:, 0:1], 1.0)
    msg = seg_ref[...] / deg
    z = (
        jnp.dot(h, ws_ref[...], preferred_element_type=jnp.float32)
        + jnp.dot(msg, wn_ref[...], preferred_element_type=jnp.float32)
        + b_ref[...]
    )
    o_ref[...] = h + jnp.where(z >= 0, z, 0.01 * z)


def _dense(h, seg, deg8, Ws, Wn, bias):
    return pl.pallas_call(
        _dense_body,
        grid=(N // BN,),
        in_specs=[
            pl.BlockSpec((BN, D), lambda i: (i, 0)),
            pl.BlockSpec((BN, D), lambda i: (i, 0)),
            pl.BlockSpec((BN, 8), lambda i: (i, 0)),
            pl.BlockSpec((D, D), lambda i: (0, 0)),
            pl.BlockSpec((D, D), lambda i: (0, 0)),
            pl.BlockSpec((1, D), lambda i: (0, 0)),
        ],
        out_specs=pl.BlockSpec((BN, D), lambda i: (i, 0)),
        out_shape=jax.ShapeDtypeStruct((N, D), jnp.float32),
    )(h, seg, deg8, Ws, Wn, bias)


def kernel(x, edge_index, W_in, b_in, Wself, Wnei, b):
    src = edge_index[0]
    dst2 = edge_index[1].reshape(EP // G, G)
    zb = jnp.zeros((ZROWS, D), jnp.float32)
    zb8 = jnp.zeros((ZROWS, 8), jnp.float32)
    ones8 = jnp.ones((G, 8), jnp.float32)

    h = _embed(x, W_in, b_in.reshape(1, D))
    deg8 = _degcount(dst2, zb8, ones8)
    for l in range(3):
        seg = _segsum(h, src, dst2, zb)
        h = _dense(h, seg, deg8, Wself[l], Wnei[l], b[l].reshape(1, D))
    return h


# speedup vs baseline: 1.2836x; 1.0167x over previous
import functools

import jax
import jax.numpy as jnp
from jax import lax
from jax.experimental import pallas as pl
from jax.experimental.pallas import tpu as pltpu
from jax.experimental.pallas import tpu_sc as plsc

N = 50000
E = 800000
D = 64
HALF = N // 2
G = 80
EP = E
EPT = EP // 16
CE = 2000
CI = CE // G
NOUT = EPT // CE
TRASH = 25088
ACC_R = TRASH + G
ZROWS = 1568
ZLAST = HALF - 15 * ZROWS
NBUF = 4
KAH = 3

_mesh = plsc.VectorSubcoreMesh(core_axis_name="c", subcore_axis_name="s")


@functools.partial(
    pl.kernel,
    out_type=jax.ShapeDtypeStruct((N, 8), jnp.float32),
    mesh=_mesh,
    compiler_params=pltpu.CompilerParams(use_tc_tiling_on_sc=False),
    scratch_types=[
        pltpu.VMEM((2, CI, G), jnp.int32),
        pltpu.VMEM((G, 8), jnp.float32),
        pltpu.VMEM_SHARED((ACC_R, 8), jnp.float32),
        pltpu.SemaphoreType.DMA,
        pltpu.SemaphoreType.DMA,
    ],
)
def _degcount(dst2, zb8, ones8, deg8, dstb, onesb, acc8, dsem, stsem):
    c = lax.axis_index("c")
    s = lax.axis_index("s")
    lo = c * HALF
    pltpu.sync_copy(ones8, onesb)

    @pl.when(s < 15)
    def _zero_main():
        pltpu.sync_copy(zb8, acc8.at[pl.ds(s * ZROWS, ZROWS)])

    @pl.when(s == 15)
    def _zero_last():
        pltpu.sync_copy(zb8.at[pl.ds(0, ZLAST)],
                        acc8.at[pl.ds(15 * ZROWS, ZLAST)])

    plsc.subcore_barrier()

    base = s * EPT

    def _stage(i, ib):
        pltpu.async_copy(dst2.at[pl.ds((base + i * CE) // G, CI)],
                         dstb.at[ib], stsem)

    _stage(0, 0)

    def outer(i, carry):
        ib = lax.rem(i, 2)
        pltpu.make_async_copy(dst2.at[pl.ds(0, CI)], dstb.at[ib],
                              stsem).wait()

        @pl.when(i + 1 < NOUT)
        def _stage_next():
            _stage(i + 1, 1 - ib)

        def comp(j, carry2):
            for k in range(G // 16):
                d = dstb[ib, j, pl.ds(k * 16, 16)]
                keep = (d >= lo) & (d < lo + HALF)
                trash = TRASH + k * 16 + lax.iota(jnp.int32, 16)
                dstb[ib, j, pl.ds(k * 16, 16)] = jnp.where(keep, d - lo,
                                                           trash)
            return carry2

        lax.fori_loop(0, CI, comp, 0)

        sd = [None] * CI
        for j in range(CI):
            sd[j] = pltpu.async_copy(onesb, acc8.at[dstb.at[ib, j]], dsem,
                                     add=True)
            if j >= 6:
                sd[j - 6].wait()
        for j in range(max(CI - 6, 0), CI):
            sd[j].wait()
        return carry

    lax.fori_loop(0, NOUT, outer, 0)
    plsc.subcore_barrier()

    @pl.when(s < 15)
    def _out_main():
        pltpu.sync_copy(acc8.at[pl.ds(s * ZROWS, ZROWS)],
                        deg8.at[pl.ds(c * HALF + s * ZROWS, ZROWS)])

    @pl.when(s == 15)
    def _out_last():
        pltpu.sync_copy(acc8.at[pl.ds(15 * ZROWS, ZLAST)],
                        deg8.at[pl.ds(c * HALF + 15 * ZROWS, ZLAST)])


SCE = 2000
SCI = SCE // G
NGRP = EPT // G
NCH = EPT // SCE


@functools.partial(
    pl.kernel,
    out_type=jax.ShapeDtypeStruct((N, D), jnp.float32),
    mesh=_mesh,
    compiler_params=pltpu.CompilerParams(use_tc_tiling_on_sc=False),
    scratch_types=[
        pltpu.VMEM((2, SCE), jnp.int32),
        pltpu.VMEM((2, SCI, G), jnp.int32),
        pltpu.VMEM((NBUF, G, D), jnp.float32),
        pltpu.VMEM_SHARED((ACC_R, D), jnp.float32),
        pltpu.SemaphoreType.DMA((NBUF,)),
        pltpu.SemaphoreType.DMA((NBUF,)),
        pltpu.SemaphoreType.DMA,
    ],
)
def _segsum(h, src, dst2, zb, seg, srcb, dstb, rows, acc, gsem, ssem, stsem):
    c = lax.axis_index("c")
    s = lax.axis_index("s")
    lo = c * HALF

    @pl.when(s < 15)
    def _zero_main():
        pltpu.sync_copy(zb, acc.at[pl.ds(s * ZROWS, ZROWS)])

    @pl.when(s == 15)
    def _zero_last():
        pltpu.sync_copy(zb.at[pl.ds(0, ZLAST)],
                        acc.at[pl.ds(15 * ZROWS, ZLAST)])

    plsc.subcore_barrier()

    base = s * EPT

    def _stage(i, ib):
        e0 = base + i * SCE
        pltpu.async_copy(src.at[pl.ds(e0, SCE)], srcb.at[ib], stsem)
        pltpu.async_copy(dst2.at[pl.ds(e0 // G, SCI)], dstb.at[ib], stsem)

    _stage(0, 0)

    def outer(i, carry):
        ib = lax.rem(i, 2)
        pltpu.make_async_copy(src.at[pl.ds(0, SCE)], srcb.at[ib],
                              stsem).wait()
        pltpu.make_async_copy(dst2.at[pl.ds(0, SCI)], dstb.at[ib],
                              stsem).wait()

        @pl.when(i + 1 < NCH)
        def _stage_next():
            _stage(i + 1, 1 - ib)

        gd = [None] * SCI
        sd = [None] * SCI
        for j in range(KAH):
            gd[j] = pltpu.async_copy(
                h.at[srcb.at[ib, pl.ds(j * G, G)]], rows.at[j % NBUF],
                gsem.at[j % NBUF])
        for j in range(SCI):
            b = j % NBUF
            for k in range(G // 16):
                d = dstb[ib, j, pl.ds(k * 16, 16)]
                keep = (d >= lo) & (d < lo + HALF)
                trash = TRASH + k * 16 + lax.iota(jnp.int32, 16)
                dstb[ib, j, pl.ds(k * 16, 16)] = jnp.where(keep, d - lo,
                                                           trash)
            gd[j].wait()
            if j >= NBUF - KAH:
                sd[j - (NBUF - KAH)].wait()
            jn = j + KAH
            if jn < SCI:
                gd[jn] = pltpu.async_copy(
                    h.at[srcb.at[ib, pl.ds(jn * G, G)]], rows.at[jn % NBUF],
                    gsem.at[jn % NBUF])
            sd[j] = pltpu.async_copy(rows.at[b], acc.at[dstb.at[ib, j]],
                                     ssem.at[b], add=True)
        for j in range(max(SCI - (NBUF - KAH), 0), SCI):
            sd[j].wait()
        return carry

    lax.fori_loop(0, NCH, outer, 0)
    plsc.subcore_barrier()

    @pl.when(s < 15)
    def _out_main():
        pltpu.sync_copy(acc.at[pl.ds(s * ZROWS, ZROWS)],
                        seg.at[pl.ds(c * HALF + s * ZROWS, ZROWS)])

    @pl.when(s == 15)
    def _out_last():
        pltpu.sync_copy(acc.at[pl.ds(15 * ZROWS, ZLAST)],
                        seg.at[pl.ds(c * HALF + 15 * ZROWS, ZLAST)])


BN = 5000


def _embed_body(x_ref, w_ref, b_ref, o_ref):
    o_ref[...] = (
        jnp.dot(x_ref[...], w_ref[...], preferred_element_type=jnp.float32)
        + b_ref[...]
    )


def _embed(x, W_in, b_in):
    return pl.pallas_call(
        _embed_body,
        grid=(N // BN,),
        in_specs=[
            pl.BlockSpec((BN, 2), lambda i: (i, 0)),
            pl.BlockSpec((2, D), lambda i: (0, 0)),
            pl.BlockSpec((1, D), lambda i: (0, 0)),
        ],
        out_specs=pl.BlockSpec((BN, D), lambda i: (i, 0)),
        out_shape=jax.ShapeDtypeStruct((N, D), jnp.float32),
    )(x, W_in, b_in)


def _dense_body(h_ref, seg_ref, deg_ref, ws_ref, wn_ref, b_ref, o_ref):
    h = h_ref[...]
    deg = jnp.maximum(deg_ref[:, 0:1], 1.0)
    msg = seg_ref[...] / deg
    z = (
        jnp.dot(h, ws_ref[...], preferred_element_type=jnp.float32)
        + jnp.dot(msg, wn_ref[...], preferred_element_type=jnp.float32)
        + b_ref[...]
    )
    o_ref[...] = h + jnp.where(z >= 0, z, 0.01 * z)


def _dense(h, seg, deg8, Ws, Wn, bias):
    return pl.pallas_call(
        _dense_body,
        grid=(N // BN,),
        in_specs=[
            pl.BlockSpec((BN, D), lambda i: (i, 0)),
            pl.BlockSpec((BN, D), lambda i: (i, 0)),
            pl.BlockSpec((BN, 8), lambda i: (i, 0)),
            pl.BlockSpec((D, D), lambda i: (0, 0)),
            pl.BlockSpec((D, D), lambda i: (0, 0)),
            pl.BlockSpec((1, D), lambda i: (0, 0)),
        ],
        out_specs=pl.BlockSpec((BN, D), lambda i: (i, 0)),
        out_shape=jax.ShapeDtypeStruct((N, D), jnp.float32),
    )(h, seg, deg8, Ws, Wn, bias)


def kernel(x, edge_index, W_in, b_in, Wself, Wnei, b):
    src = edge_index[0]
    dst2 = edge_index[1].reshape(EP // G, G)
    zb = jnp.zeros((ZROWS, D), jnp.float32)
    zb8 = jnp.zeros((ZROWS, 8), jnp.float32)
    ones8 = jnp.ones((G, 8), jnp.float32)

    h = _embed(x, W_in, b_in.reshape(1, D))
    deg8 = _degcount(dst2, zb8, ones8)
    for l in range(3):
        seg = _segsum(h, src, dst2, zb)
        h = _dense(h, seg, deg8, Wself[l], Wnei[l], b[l].reshape(1, D))
    return h
